# Initial kernel scaffold; baseline (speedup 1.0000x reference)
import jax
import jax.numpy as jnp
from jax.experimental import pallas as pl


def kernel(X, target):
    raise NotImplementedError("write your pallas kernel here")


# speedup vs baseline: 11.2742x; 11.2742x over previous
import functools

import jax
import jax.numpy as jnp
from jax import lax
from jax.experimental import pallas as pl
from jax.experimental.pallas import tpu as pltpu
from jax.experimental.pallas import tpu_sc as plsc

B = 128
N = 100000
L = 16
NV = N // L
NW = 32
ROWS_PER_W = B // NW

_mesh = plsc.VectorSubcoreMesh(core_axis_name="c", subcore_axis_name="s")


def _bcast_last(v, scratch):
    scratch[...] = v
    return plsc.load_gather(scratch, [jnp.full((L,), L - 1, jnp.int32)])


def _allsum(v, scratch):
    return _bcast_last(plsc.cumsum(v), scratch)


def _allmax(v, scratch):
    return _bcast_last(plsc.cummax(v), scratch)


@functools.partial(
    pl.kernel,
    out_type=jax.ShapeDtypeStruct((NW, L), jnp.float32),
    scratch_types=[
        pltpu.VMEM((N,), jnp.float32),
        pltpu.VMEM((B,), jnp.int32),
        pltpu.VMEM((L,), jnp.float32),
        pltpu.VMEM((L,), jnp.float32),
        pltpu.VMEM((L,), jnp.int32),
    ],
    mesh=_mesh,
    compiler_params=pltpu.CompilerParams(needs_layout_passes=False),
)
def _sc_loss(x_hbm, t_hbm, out_hbm, row_v, targ_v, acc_v, red_f, red_i):
    wid = lax.axis_index("c") * 16 + lax.axis_index("s")
    lanes = lax.iota(jnp.int32, L)
    pltpu.sync_copy(t_hbm, targ_v)

    def pass_skq(tau):
        def body(i, c):
            s, k, q = c
            v = row_v[pl.ds(i * L, L)]
            sv = jnp.where(v > tau, v, 0.0)
            return (s + sv, k + jnp.where(v > tau, 1.0, 0.0), q + sv * v)

        z = jnp.zeros((L,), jnp.float32)
        s, k, q = lax.fori_loop(0, NV, body, (z, z, z))
        return _allsum(s, red_f), _allsum(k, red_f), _allsum(q, red_f)

    acc = jnp.zeros((L,), jnp.float32)
    for r in range(ROWS_PER_W):
        row = wid * ROWS_PER_W + r
        pltpu.sync_copy(x_hbm.at[row], row_v)

        def max_body(i, m):
            return jnp.maximum(m, row_v[pl.ds(i * L, L)])

        m16 = lax.fori_loop(0, NV, max_body,
                            jnp.full((L,), -3.4e38, jnp.float32))
        big = _allmax(m16, red_f)

        blk = (row // L) * L
        tvec = targ_v[pl.ds(blk, L)]
        tg = _allsum(jnp.where(lanes == row - blk, tvec, 0), red_i)
        x_t = plsc.load_gather(row_v, [tg])

        tau0 = big - 1.0
        s, k, q = pass_skq(tau0)
        carry = (tau0, (s - 1.0) / k, jnp.int32(1), s, k, q)

        def w_cond(c):
            tau, new_tau, it, _, _, _ = c
            return jnp.all(new_tau > tau) & (it < 64)

        def w_body(c):
            _, tau, it, _, _, _ = c
            s, k, q = pass_skq(tau)
            return (tau, (s - 1.0) / k, it + 1, s, k, q)

        _, tau, _, s, k, q = lax.while_loop(w_cond, w_body, carry)

        sum_px = q - tau * s
        sum_p2 = q - 2.0 * tau * s + k * tau * tau
        loss = (1.0 - sum_p2) * 0.5 + sum_px - x_t
        acc = acc + jnp.where(lanes == r, loss, 0.0)

    acc_v[...] = acc
    pltpu.sync_copy(acc_v, out_hbm.at[wid])


def kernel(X, target):
    part = _sc_loss(X, target.astype(jnp.int32))
    return jnp.sum(part) / jnp.float32(B)


# speedup vs baseline: 31.0530x; 2.7544x over previous
import functools

import jax
import jax.numpy as jnp
from jax import lax
from jax.experimental import pallas as pl
from jax.experimental.pallas import tpu as pltpu
from jax.experimental.pallas import tpu_sc as plsc

B = 128
N = 100000
L = 16
NV = N // L
NW = 32
ROWS_PER_W = B // NW
U = 5
CAPV = 64

_mesh = plsc.VectorSubcoreMesh(core_axis_name="c", subcore_axis_name="s")


def _bcast_last(v, scratch):
    scratch[...] = v
    return plsc.load_gather(scratch, [jnp.full((L,), L - 1, jnp.int32)])


def _allsum(v, scratch):
    return _bcast_last(plsc.cumsum(v), scratch)


def _allmax(v, scratch):
    return _bcast_last(plsc.cummax(v), scratch)


@functools.partial(
    pl.kernel,
    out_type=jax.ShapeDtypeStruct((NW, L), jnp.float32),
    scratch_types=[
        pltpu.VMEM((N,), jnp.float32),
        pltpu.VMEM((CAPV * L,), jnp.float32),
        pltpu.VMEM((B,), jnp.int32),
        pltpu.VMEM((L,), jnp.float32),
        pltpu.VMEM((L,), jnp.float32),
        pltpu.VMEM((L,), jnp.int32),
    ],
    mesh=_mesh,
    compiler_params=pltpu.CompilerParams(needs_layout_passes=False),
)
def _sc_loss(x_hbm, t_hbm, out_hbm, row_v, cand_v, targ_v, acc_v, red_f,
             red_i):
    wid = lax.axis_index("c") * 16 + lax.axis_index("s")
    lanes = lax.iota(jnp.int32, L)
    pltpu.sync_copy(t_hbm, targ_v)

    def michelot(ref, nvec, tau0):
        def pass_skq(tau):
            def body(i, c):
                s, k, q = c
                v = ref[pl.ds(i * L, L)]
                sv = jnp.where(v > tau, v, 0.0)
                return (s + sv, k + jnp.where(v > tau, 1.0, 0.0), q + sv * v)

            z = jnp.zeros((L,), jnp.float32)
            s, k, q = lax.fori_loop(0, nvec, body, (z, z, z))
            return _allsum(s, red_f), _allsum(k, red_f), _allsum(q, red_f)

        s, k, q = pass_skq(tau0)
        carry = (tau0, (s - 1.0) / k, jnp.int32(1), s, k, q)

        def w_cond(c):
            tau, new_tau, it, _, _, _ = c
            return jnp.all(new_tau > tau) & (it < 64)

        def w_body(c):
            _, tau, it, _, _, _ = c
            s, k, q = pass_skq(tau)
            return (tau, (s - 1.0) / k, it + 1, s, k, q)

        _, tau, _, s, k, q = lax.while_loop(w_cond, w_body, carry)
        return tau, s, k, q

    acc = jnp.zeros((L,), jnp.float32)
    for r in range(ROWS_PER_W):
        row = wid * ROWS_PER_W + r
        pltpu.sync_copy(x_hbm.at[row], row_v)

        def max_body(i, ms):
            return tuple(
                jnp.maximum(ms[u], row_v[pl.ds(i * (U * L) + u * L, L)])
                for u in range(U))

        neg = jnp.full((L,), -jnp.inf, jnp.float32)
        ms = lax.fori_loop(0, NV // U, max_body, (neg,) * U)
        m16 = ms[0]
        for u in range(1, U):
            m16 = jnp.maximum(m16, ms[u])
        big = _allmax(m16, red_f)
        thr = big - 1.0

        def fill_body(i, z):
            cand_v[pl.ds(i * L, L)] = neg
            return z

        lax.fori_loop(0, CAPV, fill_body, 0)

        def comp_body(i, cur):
            for u in range(U):
                v = row_v[pl.ds(i * (U * L) + u * L, L)]
                keep = v > thr
                slot = jnp.minimum(cur, CAPV - 1) * L + lanes
                plsc.store_scatter(cand_v, [slot], v,
                                   mask=keep & (cur < CAPV))
                cur = cur + jnp.where(keep, 1, 0)
            return cur

        cur = lax.fori_loop(0, NV // U, comp_body,
                            jnp.zeros((L,), jnp.int32))
        overflow = jnp.any(cur > CAPV)

        blk = (row // L) * L
        tvec = targ_v[pl.ds(blk, L)]
        tg = _allsum(jnp.where(lanes == row - blk, tvec, 0), red_i)
        x_t = plsc.load_gather(row_v, [tg])

        tau, s, k, q = lax.cond(
            overflow,
            lambda: michelot(row_v, NV, thr),
            lambda: michelot(cand_v, CAPV, thr))

        sum_px = q - tau * s
        sum_p2 = q - 2.0 * tau * s + k * tau * tau
        loss = (1.0 - sum_p2) * 0.5 + sum_px - x_t
        acc = acc + jnp.where(lanes == r, loss, 0.0)

    acc_v[...] = acc
    pltpu.sync_copy(acc_v, out_hbm.at[wid])


def kernel(X, target):
    part = _sc_loss(X, target.astype(jnp.int32))
    return jnp.sum(part) / jnp.float32(B)
